# baseline (speedup 1.0000x reference)
import functools
from typing import Sequence

import jax
import jax.numpy as jnp
from jax import lax
from jax.experimental import pallas as pl
from jax.experimental.pallas import tpu as pltpu
from jax.experimental.pallas import tpu_sc as plsc

_BN_EPS = 1e-5
_NW = 32


def _pick_bm(m, target):
    for bm in (target, 2048, 1600, 1280, 1024, 1000, 800, 640, 512, 400, 320,
               256, 200, 160, 128, 80, 64, 40, 32, 16, 8):
        if bm <= m and m % bm == 0 and bm % 8 == 0:
            return bm
    return m


def _linear_call(arrs, terms, b, *, act, want_stats, bm_target=1280,
                 nsplit=1):
    m = arrs[0].shape[0]
    dout = terms[0][4].shape[1]
    dpart = dout // nsplit
    bm = _pick_bm(m, bm_target)
    grid = (m // bm,)

    n_arr = len(arrs)
    has_aff = [t[2] is not None for t in terms]

    def body(*refs):
        arr_refs = refs[:n_arr]
        k = n_arr
        term_data = []
        for (ia, ib, s, t, _w), aff in zip(terms, has_aff):
            s_ref = t_ref = None
            if aff:
                s_ref, t_ref = refs[k], refs[k + 1]
                k += 2
            w_ref = refs[k]
            k += 1
            term_data.append((ia, ib, s_ref, t_ref, w_ref))
        b_ref = refs[k]
        k += 1
        out_refs = refs[k:k + nsplit]
        st_ref = refs[k + nsplit] if want_stats else None

        acc = jnp.zeros((bm, dout), jnp.float32) + b_ref[...]
        for (ia, ib, s_ref, t_ref, w_ref) in term_data:
            x = arr_refs[ia][...]
            if ib is not None:
                x = x - arr_refs[ib][...]
            if s_ref is not None:
                x = x * s_ref[...] + t_ref[...]
            acc = acc + jnp.dot(x, w_ref[...],
                                precision=lax.Precision.DEFAULT,
                                preferred_element_type=jnp.float32)
        if act == 'relu':
            acc = jnp.maximum(acc, 0.0)
        elif act == 'sigmoid':
            acc = jax.nn.sigmoid(acc)
        for p_i, o_ref in enumerate(out_refs):
            o_ref[...] = acc[:, p_i * dpart:(p_i + 1) * dpart]
        if want_stats:
            s1 = jnp.sum(acc, axis=0, keepdims=True)
            s2 = jnp.sum(acc * acc, axis=0, keepdims=True)
            z = jnp.concatenate([s1, s2], axis=0)
            i = pl.program_id(0)

            @pl.when(i == 0)
            def _():
                st_ref[...] = z

            @pl.when(i > 0)
            def _():
                st_ref[...] += z

    in_specs = []
    inputs = []
    for a in arrs:
        inputs.append(a)
        in_specs.append(pl.BlockSpec((bm, a.shape[1]), lambda i: (i, 0)))
    for (ia, ib, s, t, w), aff in zip(terms, has_aff):
        din = w.shape[0]
        if aff:
            inputs += [s.reshape(1, din), t.reshape(1, din)]
            in_specs += [pl.BlockSpec((1, din), lambda i: (0, 0))] * 2
        inputs.append(w)
        in_specs.append(pl.BlockSpec((din, dout), lambda i: (0, 0)))
    inputs.append(b.reshape(1, dout))
    in_specs.append(pl.BlockSpec((1, dout), lambda i: (0, 0)))

    out_shape = [jax.ShapeDtypeStruct((m, dpart), jnp.float32)] * nsplit
    out_specs = [pl.BlockSpec((bm, dpart), lambda i: (i, 0))] * nsplit
    if want_stats:
        out_shape.append(jax.ShapeDtypeStruct((2, dout), jnp.float32))
        out_specs.append(pl.BlockSpec((2, dout), lambda i: (0, 0)))

    res = pl.pallas_call(
        body, grid=grid, in_specs=in_specs, out_specs=out_specs,
        out_shape=out_shape)(*inputs)
    outs = res[0] if nsplit == 1 else list(res[:nsplit])
    return (outs, res[nsplit]) if want_stats else (outs, None)


def _colstats_call(specs, *, bm_target=1280):
    m = specs[0][0].shape[0]
    bm = _pick_bm(m, bm_target)
    grid = (m // bm,)
    n_out = len(specs)

    flat = []
    layout = []
    for sp in specs:
        layout.append((len(flat), len(sp) == 2))
        flat.extend(sp)

    def body(*refs):
        in_refs = refs[:len(flat)]
        out_refs = refs[len(flat):]
        i = pl.program_id(0)
        for (start, has_b), o_ref in zip(layout, out_refs):
            x = in_refs[start][...]
            if has_b:
                x = x - in_refs[start + 1][...]
            s1 = jnp.sum(x, axis=0, keepdims=True)
            s2 = jnp.sum(x * x, axis=0, keepdims=True)
            z = jnp.concatenate([s1, s2], axis=0)

            @pl.when(i == 0)
            def _(o_ref=o_ref, z=z):
                o_ref[...] = z

            @pl.when(i > 0)
            def _(o_ref=o_ref, z=z):
                o_ref[...] += z

    in_specs = [pl.BlockSpec((bm, a.shape[1]), lambda i: (i, 0)) for a in flat]
    out_shape = [jax.ShapeDtypeStruct((2, sp[0].shape[1]), jnp.float32)
                 for sp in specs]
    out_specs = [pl.BlockSpec((2, sp[0].shape[1]), lambda i: (0, 0))
                 for sp in specs]
    res = pl.pallas_call(body, grid=grid, in_specs=in_specs,
                         out_specs=out_specs, out_shape=out_shape)(*flat)
    return list(res)


def _bn_affine(stats, m):
    mu = stats[0] / m
    var = stats[1] / m - mu * mu
    s = lax.rsqrt(var + _BN_EPS)
    return s, -mu * s


def _compose_affine(s_in, t_in, s_out, t_out):
    return s_in * s_out, t_in * s_out + t_out


def _affine_stats(stats, s, t, m):
    s1, s2 = stats[0], stats[1]
    return jnp.stack([s * s1 + m * t,
                      s * s * s2 + 2.0 * s * t * s1 + m * t * t])


def _sc_gather2(table, idx0, idx1):
    e = idx0.shape[0]
    d = table.shape[1]
    per_w = e // _NW
    r = next(c for c in (200, 80, 40, 8) if per_w % c == 0)
    n_chunks = per_w // r
    mesh = plsc.VectorSubcoreMesh(core_axis_name="c", subcore_axis_name="s")

    @functools.partial(
        pl.kernel,
        out_type=[jax.ShapeDtypeStruct((e, d), jnp.float32)] * 2,
        mesh=mesh,
        scratch_types=[
            pltpu.VMEM((per_w,), jnp.int32),
            pltpu.VMEM((r, d), jnp.float32),
            pltpu.SemaphoreType.DMA,
        ],
    )
    def k(table_hbm, i0_hbm, i1_hbm, o0_hbm, o1_hbm, idx_v, rows_v, sem):
        wid = lax.axis_index("s") * 2 + lax.axis_index("c")
        base = wid * per_w
        for i_hbm, o_hbm in ((i0_hbm, o0_hbm), (i1_hbm, o1_hbm)):
            pltpu.sync_copy(i_hbm.at[pl.ds(base, per_w)], idx_v)

            def body(c, _, o_hbm=o_hbm):
                pltpu.async_copy(
                    table_hbm.at[idx_v.at[pl.ds(c * r, r)]], rows_v,
                    sem).wait()
                pltpu.sync_copy(rows_v, o_hbm.at[pl.ds(base + c * r, r)])
                return _

            lax.fori_loop(0, n_chunks, body, 0)

    return k(table, idx0, idx1)


def _segment_max_affine(msg, dst, s, t, n_nodes):
    agg = jax.ops.segment_max(msg, dst, num_segments=n_nodes)
    return jnp.where(jnp.isfinite(agg), agg * s + t, 0.0)


def _mlp3_edge(arrs, terms_in, w1_list, p, *, e_rows, nsplit_out=1):
    b1 = p['lin1']['b']
    terms = [(ia, ib, s, t, w)
             for (ia, ib, s, t), w in zip(terms_in, w1_list)]
    y1, st1 = _linear_call(arrs, terms, b1, act='relu', want_stats=True)
    s1, t1 = _bn_affine(st1, e_rows)
    y2, st2 = _linear_call([y1], [(0, None, s1, t1, p['lin2']['W'])],
                           p['lin2']['b'], act='relu', want_stats=True)
    s2, t2 = _bn_affine(st2, e_rows)
    y3, st3 = _linear_call([y2], [(0, None, s2, t2, p['lin3']['W'])],
                           p['lin3']['b'], act='relu', want_stats=True,
                           nsplit=nsplit_out)
    s3, t3 = _bn_affine(st3, e_rows)
    return y3, (s3, t3), st3


def kernel(node_feats, edge_feats, params, edge_index):
    src = edge_index[0]
    dst = edge_index[1]
    n = node_feats.shape[0]
    e = src.shape[0]
    ef32 = jnp.float32(e)

    d0 = node_feats.shape[1]
    pad0 = (-d0) % 128
    nf = jnp.pad(node_feats, ((0, 0), (0, pad0)))
    w1n = params['nmm1']['lin1']['W']
    zpad = jnp.zeros((pad0, w1n.shape[1]), jnp.float32)
    w1n_parts = [jnp.concatenate([w1n[:d0], zpad]),
                 jnp.concatenate([w1n[d0:], zpad])]
    xd0, xs0 = _sc_gather2(nf, dst, src)
    st_a, st_b = _colstats_call([(xd0,), (xs0, xd0)])
    s0a, t0a = _bn_affine(st_a, ef32)
    s0b, t0b = _bn_affine(st_b, ef32)
    y3, (s3, t3), _ = _mlp3_edge(
        [xd0, xs0], [(0, None, s0a, t0a), (1, 0, s0b, t0b)], w1n_parts,
        params['nmm1'], e_rows=ef32)
    x1 = _segment_max_affine(y3, dst, s3, t3, n)

    xs1, xd1 = _sc_gather2(x1, src, dst)
    st_e0, st_s1, st_d1 = _colstats_call([(edge_feats,), (xs1,), (xd1,)])
    se0, te0 = _bn_affine(st_e0, ef32)
    ss1, ts1 = _bn_affine(st_s1, ef32)
    sd1, td1 = _bn_affine(st_d1, ef32)
    w1e = params['emm1']['lin1']['W']
    de0, d1 = edge_feats.shape[1], xs1.shape[1]
    e1, (es3, et3), est3 = _mlp3_edge(
        [edge_feats, xs1, xd1],
        [(0, None, se0, te0), (1, None, ss1, ts1), (2, None, sd1, td1)],
        [w1e[:de0], w1e[de0:de0 + d1], w1e[de0 + d1:]],
        params['emm1'], e_rows=ef32)

    w1n2 = params['nmm2']['lin1']['W']
    z3, (zs3, zt3), _ = _mlp3_edge(
        [xd1, xs1], [(0, None, None, None), (1, 0, None, None)],
        [w1n2[:d1], w1n2[d1:]], params['nmm2'], e_rows=ef32)
    x2 = _segment_max_affine(z3, dst, zs3, zt3, n)

    xs2, xd2 = _sc_gather2(x2, src, dst)
    st_s2, st_d2 = _colstats_call([(xs2,), (xd2,)])
    st_e1p = _affine_stats(est3, es3, et3, ef32)
    se1, te1 = _bn_affine(st_e1p, ef32)
    se1c, te1c = _compose_affine(es3, et3, se1, te1)
    ss2, ts2 = _bn_affine(st_s2, ef32)
    sd2, td2 = _bn_affine(st_d2, ef32)
    w1e2 = params['emm2']['lin1']['W']
    de1, d2 = e1.shape[1], xs2.shape[1]
    e2, (fs3, ft3), _ = _mlp3_edge(
        [e1, xs2, xd2],
        [(0, None, se1c, te1c), (1, None, ss2, ts2), (2, None, sd2, td2)],
        [w1e2[:de1], w1e2[de1:de1 + d2], w1e2[de1 + d2:]],
        params['emm2'], e_rows=ef32)

    ph = params['nhead']
    h1, _ = _linear_call([x2], [(0, None, None, None, ph['l1']['W'])],
                         ph['l1']['b'], act='relu', want_stats=False,
                         bm_target=1000)
    h2, _ = _linear_call([h1], [(0, None, None, None, ph['l2']['W'])],
                         ph['l2']['b'], act='relu', want_stats=False,
                         bm_target=1000)
    w34 = ph['l3']['W'] @ ph['l4']['W']
    b34 = ph['l3']['b'] @ ph['l4']['W'] + ph['l4']['b']
    n_out, _ = _linear_call([h2], [(0, None, None, None, w34)], b34,
                            act='sigmoid', want_stats=False, bm_target=1000)

    pe = params['ehead']
    w12 = pe['l1']['W'] @ pe['l2']['W']
    b12 = pe['l1']['b'] @ pe['l2']['W'] + pe['l2']['b']
    w12f = fs3.reshape(-1, 1) * w12
    b12f = ft3 @ w12 + b12
    g1, _ = _linear_call([e2], [(0, None, None, None, w12f)], b12f,
                         act='relu', want_stats=False)
    g2, _ = _linear_call([g1], [(0, None, None, None, pe['l3']['W'])],
                         pe['l3']['b'], act='relu', want_stats=False)
    w45 = pe['l4']['W'] @ pe['l5']['W']
    b45 = pe['l4']['b'] @ pe['l5']['W'] + pe['l5']['b']
    e_out, _ = _linear_call([g2], [(0, None, None, None, w45)], b45,
                            act='sigmoid', want_stats=False)

    return (n_out, e_out)


# speedup vs baseline: 1.0008x; 1.0008x over previous
import functools
from typing import Sequence

import jax
import jax.numpy as jnp
from jax import lax
from jax.experimental import pallas as pl
from jax.experimental.pallas import tpu as pltpu
from jax.experimental.pallas import tpu_sc as plsc

_BN_EPS = 1e-5
_NW = 32


def _pick_bm(m, target):
    for bm in (target, 2048, 1600, 1280, 1024, 1000, 800, 640, 512, 400, 320,
               256, 200, 160, 128, 80, 64, 40, 32, 16, 8):
        if bm <= m and m % bm == 0 and bm % 8 == 0:
            return bm
    return m


def _linear_call(arrs, terms, b, *, act, want_stats, bm_target=1280,
                 nsplit=1):
    m = arrs[0].shape[0]
    dout = terms[0][4].shape[1]
    dpart = dout // nsplit
    bm = _pick_bm(m, bm_target)
    grid = (m // bm,)

    n_arr = len(arrs)
    has_aff = [t[2] is not None for t in terms]

    def body(*refs):
        arr_refs = refs[:n_arr]
        k = n_arr
        term_data = []
        for (ia, ib, s, t, _w), aff in zip(terms, has_aff):
            s_ref = t_ref = None
            if aff:
                s_ref, t_ref = refs[k], refs[k + 1]
                k += 2
            w_ref = refs[k]
            k += 1
            term_data.append((ia, ib, s_ref, t_ref, w_ref))
        b_ref = refs[k]
        k += 1
        out_refs = refs[k:k + nsplit]
        st_ref = refs[k + nsplit] if want_stats else None

        acc = jnp.zeros((bm, dout), jnp.float32) + b_ref[...]
        for (ia, ib, s_ref, t_ref, w_ref) in term_data:
            x = arr_refs[ia][...]
            if ib is not None:
                x = x - arr_refs[ib][...]
            if s_ref is not None:
                x = x * s_ref[...] + t_ref[...]
            acc = acc + jnp.dot(x, w_ref[...],
                                preferred_element_type=jnp.float32)
        if act == 'relu':
            acc = jnp.maximum(acc, 0.0)
        elif act == 'sigmoid':
            acc = jax.nn.sigmoid(acc)
        for p_i, o_ref in enumerate(out_refs):
            o_ref[...] = acc[:, p_i * dpart:(p_i + 1) * dpart]
        if want_stats:
            s1 = jnp.sum(acc, axis=0, keepdims=True)
            s2 = jnp.sum(acc * acc, axis=0, keepdims=True)
            z = jnp.concatenate([s1, s2], axis=0)
            i = pl.program_id(0)

            @pl.when(i == 0)
            def _():
                st_ref[...] = z

            @pl.when(i > 0)
            def _():
                st_ref[...] += z

    in_specs = []
    inputs = []
    for a in arrs:
        inputs.append(a)
        in_specs.append(pl.BlockSpec((bm, a.shape[1]), lambda i: (i, 0)))
    for (ia, ib, s, t, w), aff in zip(terms, has_aff):
        din = w.shape[0]
        if aff:
            inputs += [s.reshape(1, din), t.reshape(1, din)]
            in_specs += [pl.BlockSpec((1, din), lambda i: (0, 0))] * 2
        inputs.append(w)
        in_specs.append(pl.BlockSpec((din, dout), lambda i: (0, 0)))
    inputs.append(b.reshape(1, dout))
    in_specs.append(pl.BlockSpec((1, dout), lambda i: (0, 0)))

    out_shape = [jax.ShapeDtypeStruct((m, dpart), jnp.float32)] * nsplit
    out_specs = [pl.BlockSpec((bm, dpart), lambda i: (i, 0))] * nsplit
    if want_stats:
        out_shape.append(jax.ShapeDtypeStruct((2, dout), jnp.float32))
        out_specs.append(pl.BlockSpec((2, dout), lambda i: (0, 0)))

    res = pl.pallas_call(
        body, grid=grid, in_specs=in_specs, out_specs=out_specs,
        out_shape=out_shape)(*inputs)
    outs = res[0] if nsplit == 1 else list(res[:nsplit])
    return (outs, res[nsplit]) if want_stats else (outs, None)


def _colstats_call(specs, *, bm_target=1280):
    m = specs[0][0].shape[0]
    bm = _pick_bm(m, bm_target)
    grid = (m // bm,)
    n_out = len(specs)

    flat = []
    layout = []
    for sp in specs:
        layout.append((len(flat), len(sp) == 2))
        flat.extend(sp)

    def body(*refs):
        in_refs = refs[:len(flat)]
        out_refs = refs[len(flat):]
        i = pl.program_id(0)
        for (start, has_b), o_ref in zip(layout, out_refs):
            x = in_refs[start][...]
            if has_b:
                x = x - in_refs[start + 1][...]
            s1 = jnp.sum(x, axis=0, keepdims=True)
            s2 = jnp.sum(x * x, axis=0, keepdims=True)
            z = jnp.concatenate([s1, s2], axis=0)

            @pl.when(i == 0)
            def _(o_ref=o_ref, z=z):
                o_ref[...] = z

            @pl.when(i > 0)
            def _(o_ref=o_ref, z=z):
                o_ref[...] += z

    in_specs = [pl.BlockSpec((bm, a.shape[1]), lambda i: (i, 0)) for a in flat]
    out_shape = [jax.ShapeDtypeStruct((2, sp[0].shape[1]), jnp.float32)
                 for sp in specs]
    out_specs = [pl.BlockSpec((2, sp[0].shape[1]), lambda i: (0, 0))
                 for sp in specs]
    res = pl.pallas_call(body, grid=grid, in_specs=in_specs,
                         out_specs=out_specs, out_shape=out_shape)(*flat)
    return list(res)


def _bn_affine(stats, m):
    mu = stats[0] / m
    var = stats[1] / m - mu * mu
    s = lax.rsqrt(var + _BN_EPS)
    return s, -mu * s


def _compose_affine(s_in, t_in, s_out, t_out):
    return s_in * s_out, t_in * s_out + t_out


def _affine_stats(stats, s, t, m):
    s1, s2 = stats[0], stats[1]
    return jnp.stack([s * s1 + m * t,
                      s * s * s2 + 2.0 * s * t * s1 + m * t * t])


def _sc_gather2(table, idx0, idx1):
    e = idx0.shape[0]
    d = table.shape[1]
    per_w = e // _NW
    r = next(c for c in (200, 80, 40, 8) if per_w % c == 0)
    n_chunks = per_w // r
    mesh = plsc.VectorSubcoreMesh(core_axis_name="c", subcore_axis_name="s")

    @functools.partial(
        pl.kernel,
        out_type=[jax.ShapeDtypeStruct((e, d), jnp.float32)] * 2,
        mesh=mesh,
        scratch_types=[
            pltpu.VMEM((per_w,), jnp.int32),
            pltpu.VMEM((r, d), jnp.float32),
            pltpu.SemaphoreType.DMA,
        ],
    )
    def k(table_hbm, i0_hbm, i1_hbm, o0_hbm, o1_hbm, idx_v, rows_v, sem):
        wid = lax.axis_index("s") * 2 + lax.axis_index("c")
        base = wid * per_w
        for i_hbm, o_hbm in ((i0_hbm, o0_hbm), (i1_hbm, o1_hbm)):
            pltpu.sync_copy(i_hbm.at[pl.ds(base, per_w)], idx_v)

            def body(c, _, o_hbm=o_hbm):
                pltpu.async_copy(
                    table_hbm.at[idx_v.at[pl.ds(c * r, r)]], rows_v,
                    sem).wait()
                pltpu.sync_copy(rows_v, o_hbm.at[pl.ds(base + c * r, r)])
                return _

            lax.fori_loop(0, n_chunks, body, 0)

    return k(table, idx0, idx1)


def _segment_max_affine(msg, dst, s, t, n_nodes):
    agg = jax.ops.segment_max(msg, dst, num_segments=n_nodes)
    return jnp.where(jnp.isfinite(agg), agg * s + t, 0.0)


def _mlp3_edge(arrs, terms_in, w1_list, p, *, e_rows, nsplit_out=1):
    b1 = p['lin1']['b']
    terms = [(ia, ib, s, t, w)
             for (ia, ib, s, t), w in zip(terms_in, w1_list)]
    y1, st1 = _linear_call(arrs, terms, b1, act='relu', want_stats=True)
    s1, t1 = _bn_affine(st1, e_rows)
    y2, st2 = _linear_call([y1], [(0, None, s1, t1, p['lin2']['W'])],
                           p['lin2']['b'], act='relu', want_stats=True)
    s2, t2 = _bn_affine(st2, e_rows)
    y3, st3 = _linear_call([y2], [(0, None, s2, t2, p['lin3']['W'])],
                           p['lin3']['b'], act='relu', want_stats=True,
                           nsplit=nsplit_out)
    s3, t3 = _bn_affine(st3, e_rows)
    return y3, (s3, t3), st3


def kernel(node_feats, edge_feats, params, edge_index):
    src = edge_index[0]
    dst = edge_index[1]
    n = node_feats.shape[0]
    e = src.shape[0]
    ef32 = jnp.float32(e)

    d0 = node_feats.shape[1]
    pad0 = (-d0) % 128
    nf = jnp.pad(node_feats, ((0, 0), (0, pad0)))
    w1n = params['nmm1']['lin1']['W']
    zpad = jnp.zeros((pad0, w1n.shape[1]), jnp.float32)
    w1n_parts = [jnp.concatenate([w1n[:d0], zpad]),
                 jnp.concatenate([w1n[d0:], zpad])]
    xd0, xs0 = _sc_gather2(nf, dst, src)
    st_a, st_b = _colstats_call([(xd0,), (xs0, xd0)])
    s0a, t0a = _bn_affine(st_a, ef32)
    s0b, t0b = _bn_affine(st_b, ef32)
    y3, (s3, t3), _ = _mlp3_edge(
        [xd0, xs0], [(0, None, s0a, t0a), (1, 0, s0b, t0b)], w1n_parts,
        params['nmm1'], e_rows=ef32)
    x1 = _segment_max_affine(y3, dst, s3, t3, n)

    xs1, xd1 = _sc_gather2(x1, src, dst)
    st_e0, st_s1, st_d1 = _colstats_call([(edge_feats,), (xs1,), (xd1,)])
    se0, te0 = _bn_affine(st_e0, ef32)
    ss1, ts1 = _bn_affine(st_s1, ef32)
    sd1, td1 = _bn_affine(st_d1, ef32)
    w1e = params['emm1']['lin1']['W']
    de0, d1 = edge_feats.shape[1], xs1.shape[1]
    e1, (es3, et3), est3 = _mlp3_edge(
        [edge_feats, xs1, xd1],
        [(0, None, se0, te0), (1, None, ss1, ts1), (2, None, sd1, td1)],
        [w1e[:de0], w1e[de0:de0 + d1], w1e[de0 + d1:]],
        params['emm1'], e_rows=ef32)

    w1n2 = params['nmm2']['lin1']['W']
    z3, (zs3, zt3), _ = _mlp3_edge(
        [xd1, xs1], [(0, None, None, None), (1, 0, None, None)],
        [w1n2[:d1], w1n2[d1:]], params['nmm2'], e_rows=ef32)
    x2 = _segment_max_affine(z3, dst, zs3, zt3, n)

    xs2, xd2 = _sc_gather2(x2, src, dst)
    st_s2, st_d2 = _colstats_call([(xs2,), (xd2,)])
    st_e1p = _affine_stats(est3, es3, et3, ef32)
    se1, te1 = _bn_affine(st_e1p, ef32)
    se1c, te1c = _compose_affine(es3, et3, se1, te1)
    ss2, ts2 = _bn_affine(st_s2, ef32)
    sd2, td2 = _bn_affine(st_d2, ef32)
    w1e2 = params['emm2']['lin1']['W']
    de1, d2 = e1.shape[1], xs2.shape[1]
    e2, (fs3, ft3), _ = _mlp3_edge(
        [e1, xs2, xd2],
        [(0, None, se1c, te1c), (1, None, ss2, ts2), (2, None, sd2, td2)],
        [w1e2[:de1], w1e2[de1:de1 + d2], w1e2[de1 + d2:]],
        params['emm2'], e_rows=ef32)

    ph = params['nhead']
    h1, _ = _linear_call([x2], [(0, None, None, None, ph['l1']['W'])],
                         ph['l1']['b'], act='relu', want_stats=False,
                         bm_target=1000)
    h2, _ = _linear_call([h1], [(0, None, None, None, ph['l2']['W'])],
                         ph['l2']['b'], act='relu', want_stats=False,
                         bm_target=1000)
    w34 = ph['l3']['W'] @ ph['l4']['W']
    b34 = ph['l3']['b'] @ ph['l4']['W'] + ph['l4']['b']
    n_out, _ = _linear_call([h2], [(0, None, None, None, w34)], b34,
                            act='sigmoid', want_stats=False, bm_target=1000)

    pe = params['ehead']
    w12 = pe['l1']['W'] @ pe['l2']['W']
    b12 = pe['l1']['b'] @ pe['l2']['W'] + pe['l2']['b']
    w12f = fs3.reshape(-1, 1) * w12
    b12f = ft3 @ w12 + b12
    g1, _ = _linear_call([e2], [(0, None, None, None, w12f)], b12f,
                         act='relu', want_stats=False)
    g2, _ = _linear_call([g1], [(0, None, None, None, pe['l3']['W'])],
                         pe['l3']['b'], act='relu', want_stats=False)
    w45 = pe['l4']['W'] @ pe['l5']['W']
    b45 = pe['l4']['b'] @ pe['l5']['W'] + pe['l5']['b']
    e_out, _ = _linear_call([g2], [(0, None, None, None, w45)], b45,
                            act='sigmoid', want_stats=False)

    return (n_out, e_out)


# speedup vs baseline: 1.0064x; 1.0056x over previous
import functools
from typing import Sequence

import jax
import jax.numpy as jnp
from jax import lax
from jax.experimental import pallas as pl
from jax.experimental.pallas import tpu as pltpu
from jax.experimental.pallas import tpu_sc as plsc

_BN_EPS = 1e-5
_NW = 32


def _pick_bm(m, target):
    for bm in (target, 2048, 1600, 1280, 1024, 1000, 800, 640, 512, 400, 320,
               256, 200, 160, 128, 80, 64, 40, 32, 16, 8):
        if bm <= m and m % bm == 0 and bm % 8 == 0:
            return bm
    return m


def _linear_call(arrs, terms, b, *, act, want_stats, bm_target=1280,
                 nsplit=1):
    m = arrs[0].shape[0]
    dout = terms[0][4].shape[1]
    dpart = dout // nsplit
    bm = _pick_bm(m, bm_target)
    grid = (m // bm,)

    n_arr = len(arrs)
    has_aff = [t[2] is not None for t in terms]

    def body(*refs):
        arr_refs = refs[:n_arr]
        k = n_arr
        term_data = []
        for (ia, ib, s, t, _w), aff in zip(terms, has_aff):
            s_ref = t_ref = None
            if aff:
                s_ref, t_ref = refs[k], refs[k + 1]
                k += 2
            w_ref = refs[k]
            k += 1
            term_data.append((ia, ib, s_ref, t_ref, w_ref))
        b_ref = refs[k]
        k += 1
        out_refs = refs[k:k + nsplit]
        st_ref = refs[k + nsplit] if want_stats else None

        acc = jnp.zeros((bm, dout), jnp.float32) + b_ref[...]
        for (ia, ib, s_ref, t_ref, w_ref) in term_data:
            x = arr_refs[ia][...]
            if ib is not None:
                x = x - arr_refs[ib][...]
            if s_ref is not None:
                x = x * s_ref[...] + t_ref[...]
            acc = acc + jnp.dot(x, w_ref[...],
                                preferred_element_type=jnp.float32)
        if act == 'relu':
            acc = jnp.maximum(acc, 0.0)
        elif act == 'sigmoid':
            acc = jax.nn.sigmoid(acc)
        for p_i, o_ref in enumerate(out_refs):
            o_ref[...] = acc[:, p_i * dpart:(p_i + 1) * dpart]
        if want_stats:
            s1 = jnp.sum(acc, axis=0, keepdims=True)
            s2 = jnp.sum(acc * acc, axis=0, keepdims=True)
            z = jnp.concatenate([s1, s2], axis=0)
            i = pl.program_id(0)

            @pl.when(i == 0)
            def _():
                st_ref[...] = z

            @pl.when(i > 0)
            def _():
                st_ref[...] += z

    in_specs = []
    inputs = []
    for a in arrs:
        inputs.append(a)
        in_specs.append(pl.BlockSpec((bm, a.shape[1]), lambda i: (i, 0)))
    for (ia, ib, s, t, w), aff in zip(terms, has_aff):
        din = w.shape[0]
        if aff:
            inputs += [s.reshape(1, din), t.reshape(1, din)]
            in_specs += [pl.BlockSpec((1, din), lambda i: (0, 0))] * 2
        inputs.append(w)
        in_specs.append(pl.BlockSpec((din, dout), lambda i: (0, 0)))
    inputs.append(b.reshape(1, dout))
    in_specs.append(pl.BlockSpec((1, dout), lambda i: (0, 0)))

    out_shape = [jax.ShapeDtypeStruct((m, dpart), jnp.float32)] * nsplit
    out_specs = [pl.BlockSpec((bm, dpart), lambda i: (i, 0))] * nsplit
    if want_stats:
        out_shape.append(jax.ShapeDtypeStruct((2, dout), jnp.float32))
        out_specs.append(pl.BlockSpec((2, dout), lambda i: (0, 0)))

    res = pl.pallas_call(
        body, grid=grid, in_specs=in_specs, out_specs=out_specs,
        out_shape=out_shape)(*inputs)
    outs = res[0] if nsplit == 1 else list(res[:nsplit])
    return (outs, res[nsplit]) if want_stats else (outs, None)


def _colstats_call(specs, *, bm_target=1280):
    m = specs[0][0].shape[0]
    bm = _pick_bm(m, bm_target)
    grid = (m // bm,)
    n_out = len(specs)

    flat = []
    layout = []
    for sp in specs:
        layout.append((len(flat), len(sp) == 2))
        flat.extend(sp)

    def body(*refs):
        in_refs = refs[:len(flat)]
        out_refs = refs[len(flat):]
        i = pl.program_id(0)
        for (start, has_b), o_ref in zip(layout, out_refs):
            x = in_refs[start][...]
            if has_b:
                x = x - in_refs[start + 1][...]
            s1 = jnp.sum(x, axis=0, keepdims=True)
            s2 = jnp.sum(x * x, axis=0, keepdims=True)
            z = jnp.concatenate([s1, s2], axis=0)

            @pl.when(i == 0)
            def _(o_ref=o_ref, z=z):
                o_ref[...] = z

            @pl.when(i > 0)
            def _(o_ref=o_ref, z=z):
                o_ref[...] += z

    in_specs = [pl.BlockSpec((bm, a.shape[1]), lambda i: (i, 0)) for a in flat]
    out_shape = [jax.ShapeDtypeStruct((2, sp[0].shape[1]), jnp.float32)
                 for sp in specs]
    out_specs = [pl.BlockSpec((2, sp[0].shape[1]), lambda i: (0, 0))
                 for sp in specs]
    res = pl.pallas_call(body, grid=grid, in_specs=in_specs,
                         out_specs=out_specs, out_shape=out_shape)(*flat)
    return list(res)


def _bn_affine(stats, m):
    mu = stats[0] / m
    var = stats[1] / m - mu * mu
    s = lax.rsqrt(var + _BN_EPS)
    return s, -mu * s


def _compose_affine(s_in, t_in, s_out, t_out):
    return s_in * s_out, t_in * s_out + t_out


def _affine_stats(stats, s, t, m):
    s1, s2 = stats[0], stats[1]
    return jnp.stack([s * s1 + m * t,
                      s * s * s2 + 2.0 * s * t * s1 + m * t * t])


def _sc_gather2(table0, idx0, idx1, table1=None):
    if table1 is None:
        table1 = table0
    e = idx0.shape[0]
    d = table0.shape[1]
    per_w = e // _NW
    r = 200 if d <= 256 else 40
    while per_w % r:
        r //= 5
    n_chunks = per_w // r
    mesh = plsc.VectorSubcoreMesh(core_axis_name="c", subcore_axis_name="s")

    @functools.partial(
        pl.kernel,
        out_type=[jax.ShapeDtypeStruct((e, d), jnp.float32)] * 2,
        mesh=mesh,
        scratch_types=[
            pltpu.VMEM((per_w,), jnp.int32),
            pltpu.VMEM((r, d), jnp.float32),
            pltpu.VMEM((r, d), jnp.float32),
            pltpu.SemaphoreType.DMA,
            pltpu.SemaphoreType.DMA,
        ],
    )
    def k(t0_hbm, t1_hbm, i0_hbm, i1_hbm, o0_hbm, o1_hbm, idx_v,
          buf0, buf1, sem0, sem1):
        wid = lax.axis_index("s") * 2 + lax.axis_index("c")
        base = wid * per_w
        for t_hbm, i_hbm, o_hbm in ((t0_hbm, i0_hbm, o0_hbm),
                                    (t1_hbm, i1_hbm, o1_hbm)):
            pltpu.sync_copy(i_hbm.at[pl.ds(base, per_w)], idx_v)

            def mk(c, buf, sem, t_hbm=t_hbm):
                return pltpu.make_async_copy(
                    t_hbm.at[idx_v.at[pl.ds(c * r, r)]], buf, sem)

            mk(0, buf0, sem0).start()

            def body(c, carry, o_hbm=o_hbm, mk=mk):
                @pl.when(c % 2 == 0)
                def _():
                    mk(c, buf0, sem0).wait()

                    @pl.when(c + 1 < n_chunks)
                    def _():
                        mk(c + 1, buf1, sem1).start()

                    pltpu.sync_copy(buf0, o_hbm.at[pl.ds(base + c * r, r)])

                @pl.when(c % 2 == 1)
                def _():
                    mk(c, buf1, sem1).wait()

                    @pl.when(c + 1 < n_chunks)
                    def _():
                        mk(c + 1, buf0, sem0).start()

                    pltpu.sync_copy(buf1, o_hbm.at[pl.ds(base + c * r, r)])

                return carry

            lax.fori_loop(0, n_chunks, body, 0)

    return k(table0, table1, idx0, idx1)


def _segment_max_affine(msg, dst, s, t, n_nodes):
    agg = jax.ops.segment_max(msg, dst, num_segments=n_nodes)
    return jnp.where(jnp.isfinite(agg), agg * s + t, 0.0)


def _mlp3_edge(arrs, terms_in, w1_list, p, *, e_rows, nsplit_out=1):
    b1 = p['lin1']['b']
    terms = [(ia, ib, s, t, w)
             for (ia, ib, s, t), w in zip(terms_in, w1_list)]
    y1, st1 = _linear_call(arrs, terms, b1, act='relu', want_stats=True)
    s1, t1 = _bn_affine(st1, e_rows)
    y2, st2 = _linear_call([y1], [(0, None, s1, t1, p['lin2']['W'])],
                           p['lin2']['b'], act='relu', want_stats=True)
    s2, t2 = _bn_affine(st2, e_rows)
    y3, st3 = _linear_call([y2], [(0, None, s2, t2, p['lin3']['W'])],
                           p['lin3']['b'], act='relu', want_stats=True,
                           nsplit=nsplit_out)
    s3, t3 = _bn_affine(st3, e_rows)
    return y3, (s3, t3), st3


def kernel(node_feats, edge_feats, params, edge_index):
    src = edge_index[0]
    dst = edge_index[1]
    n = node_feats.shape[0]
    e = src.shape[0]
    ef32 = jnp.float32(e)

    d0 = node_feats.shape[1]
    pad0 = (-d0) % 128
    nf = jnp.pad(node_feats, ((0, 0), (0, pad0)))
    w1n = params['nmm1']['lin1']['W']
    zpad = jnp.zeros((pad0, w1n.shape[1]), jnp.float32)
    w1n_parts = [jnp.concatenate([w1n[:d0], zpad]),
                 jnp.concatenate([w1n[d0:], zpad])]
    xd0, xs0 = _sc_gather2(nf, dst, src)
    st_a, st_b = _colstats_call([(xd0,), (xs0, xd0)])
    s0a, t0a = _bn_affine(st_a, ef32)
    s0b, t0b = _bn_affine(st_b, ef32)
    y3, (s3, t3), _ = _mlp3_edge(
        [xd0, xs0], [(0, None, s0a, t0a), (1, 0, s0b, t0b)], w1n_parts,
        params['nmm1'], e_rows=ef32)
    x1 = _segment_max_affine(y3, dst, s3, t3, n)

    xs1, xd1 = _sc_gather2(x1, src, dst)
    st_e0, st_s1, st_d1 = _colstats_call([(edge_feats,), (xs1,), (xd1,)])
    se0, te0 = _bn_affine(st_e0, ef32)
    ss1, ts1 = _bn_affine(st_s1, ef32)
    sd1, td1 = _bn_affine(st_d1, ef32)
    w1e = params['emm1']['lin1']['W']
    de0, d1 = edge_feats.shape[1], xs1.shape[1]
    e1, (es3, et3), est3 = _mlp3_edge(
        [edge_feats, xs1, xd1],
        [(0, None, se0, te0), (1, None, ss1, ts1), (2, None, sd1, td1)],
        [w1e[:de0], w1e[de0:de0 + d1], w1e[de0 + d1:]],
        params['emm1'], e_rows=ef32)

    w1n2 = params['nmm2']['lin1']['W']
    z3, (zs3, zt3), _ = _mlp3_edge(
        [xd1, xs1], [(0, None, None, None), (1, 0, None, None)],
        [w1n2[:d1], w1n2[d1:]], params['nmm2'], e_rows=ef32)
    x2 = _segment_max_affine(z3, dst, zs3, zt3, n)

    xs2, xd2 = _sc_gather2(x2, src, dst)
    st_s2, st_d2 = _colstats_call([(xs2,), (xd2,)])
    st_e1p = _affine_stats(est3, es3, et3, ef32)
    se1, te1 = _bn_affine(st_e1p, ef32)
    se1c, te1c = _compose_affine(es3, et3, se1, te1)
    ss2, ts2 = _bn_affine(st_s2, ef32)
    sd2, td2 = _bn_affine(st_d2, ef32)
    w1e2 = params['emm2']['lin1']['W']
    de1, d2 = e1.shape[1], xs2.shape[1]
    e2, (fs3, ft3), _ = _mlp3_edge(
        [e1, xs2, xd2],
        [(0, None, se1c, te1c), (1, None, ss2, ts2), (2, None, sd2, td2)],
        [w1e2[:de1], w1e2[de1:de1 + d2], w1e2[de1 + d2:]],
        params['emm2'], e_rows=ef32)

    ph = params['nhead']
    h1, _ = _linear_call([x2], [(0, None, None, None, ph['l1']['W'])],
                         ph['l1']['b'], act='relu', want_stats=False,
                         bm_target=1000)
    h2, _ = _linear_call([h1], [(0, None, None, None, ph['l2']['W'])],
                         ph['l2']['b'], act='relu', want_stats=False,
                         bm_target=1000)
    w34 = ph['l3']['W'] @ ph['l4']['W']
    b34 = ph['l3']['b'] @ ph['l4']['W'] + ph['l4']['b']
    n_out, _ = _linear_call([h2], [(0, None, None, None, w34)], b34,
                            act='sigmoid', want_stats=False, bm_target=1000)

    pe = params['ehead']
    w12 = pe['l1']['W'] @ pe['l2']['W']
    b12 = pe['l1']['b'] @ pe['l2']['W'] + pe['l2']['b']
    w12f = fs3.reshape(-1, 1) * w12
    b12f = ft3 @ w12 + b12
    g1, _ = _linear_call([e2], [(0, None, None, None, w12f)], b12f,
                         act='relu', want_stats=False)
    g2, _ = _linear_call([g1], [(0, None, None, None, pe['l3']['W'])],
                         pe['l3']['b'], act='relu', want_stats=False)
    w45 = pe['l4']['W'] @ pe['l5']['W']
    b45 = pe['l4']['b'] @ pe['l5']['W'] + pe['l5']['b']
    e_out, _ = _linear_call([g2], [(0, None, None, None, w45)], b45,
                            act='sigmoid', want_stats=False)

    return (n_out, e_out)


# speedup vs baseline: 1.0764x; 1.0696x over previous
import functools
from typing import Sequence

import jax
import jax.numpy as jnp
from jax import lax
from jax.experimental import pallas as pl
from jax.experimental.pallas import tpu as pltpu
from jax.experimental.pallas import tpu_sc as plsc

_BN_EPS = 1e-5
_NW = 32


def _pick_bm(m, target):
    for bm in (target, 2048, 1600, 1280, 1024, 1000, 800, 640, 512, 400, 320,
               256, 200, 160, 128, 80, 64, 40, 32, 16, 8):
        if bm <= m and m % bm == 0 and bm % 8 == 0:
            return bm
    return m


def _linear_call(arrs, terms, b, *, act, want_stats, bm_target=1280,
                 nsplit=1):
    m = arrs[0].shape[0]
    dout = terms[0][4].shape[1]
    dpart = dout // nsplit
    bm = _pick_bm(m, bm_target)
    grid = (m // bm,)

    n_arr = len(arrs)
    has_aff = [t[2] is not None for t in terms]

    def body(*refs):
        arr_refs = refs[:n_arr]
        k = n_arr
        term_data = []
        for (ia, ib, s, t, w), aff in zip(terms, has_aff):
            s_ref = t_ref = None
            if aff:
                s_ref, t_ref = refs[k], refs[k + 1]
                k += 2
            w_ref = None
            if w is not None:
                w_ref = refs[k]
                k += 1
            term_data.append((ia, ib, s_ref, t_ref, w_ref))
        b_ref = refs[k]
        k += 1
        out_refs = refs[k:k + nsplit]
        st_ref = refs[k + nsplit] if want_stats else None

        acc = jnp.zeros((bm, dout), jnp.float32) + b_ref[...]
        for (ia, ib, s_ref, t_ref, w_ref) in term_data:
            x = arr_refs[ia][...]
            if ib is not None:
                x = x - arr_refs[ib][...]
            if s_ref is not None:
                x = x * s_ref[...] + t_ref[...]
            if w_ref is None:
                acc = acc + x
            else:
                acc = acc + jnp.dot(x, w_ref[...],
                                    preferred_element_type=jnp.float32)
        if act == 'relu':
            acc = jnp.maximum(acc, 0.0)
        elif act == 'sigmoid':
            acc = jax.nn.sigmoid(acc)
        for p_i, o_ref in enumerate(out_refs):
            o_ref[...] = acc[:, p_i * dpart:(p_i + 1) * dpart]
        if want_stats:
            s1 = jnp.sum(acc, axis=0, keepdims=True)
            s2 = jnp.sum(acc * acc, axis=0, keepdims=True)
            z = jnp.concatenate([s1, s2], axis=0)
            i = pl.program_id(0)

            @pl.when(i == 0)
            def _():
                st_ref[...] = z

            @pl.when(i > 0)
            def _():
                st_ref[...] += z

    in_specs = []
    inputs = []
    for a in arrs:
        inputs.append(a)
        in_specs.append(pl.BlockSpec((bm, a.shape[1]), lambda i: (i, 0)))
    for (ia, ib, s, t, w), aff in zip(terms, has_aff):
        din = w.shape[0] if w is not None else arrs[ia].shape[1]
        if aff:
            inputs += [s.reshape(1, din), t.reshape(1, din)]
            in_specs += [pl.BlockSpec((1, din), lambda i: (0, 0))] * 2
        if w is not None:
            inputs.append(w)
            in_specs.append(pl.BlockSpec((din, dout), lambda i: (0, 0)))
    inputs.append(b.reshape(1, dout))
    in_specs.append(pl.BlockSpec((1, dout), lambda i: (0, 0)))

    out_shape = [jax.ShapeDtypeStruct((m, dpart), jnp.float32)] * nsplit
    out_specs = [pl.BlockSpec((bm, dpart), lambda i: (i, 0))] * nsplit
    if want_stats:
        out_shape.append(jax.ShapeDtypeStruct((2, dout), jnp.float32))
        out_specs.append(pl.BlockSpec((2, dout), lambda i: (0, 0)))

    res = pl.pallas_call(
        body, grid=grid, in_specs=in_specs, out_specs=out_specs,
        out_shape=out_shape)(*inputs)
    outs = res[0] if nsplit == 1 else list(res[:nsplit])
    return (outs, res[nsplit]) if want_stats else (outs, None)


def _colstats_call(specs, *, bm_target=1280):
    m = specs[0][0].shape[0]
    bm = _pick_bm(m, bm_target)
    grid = (m // bm,)
    n_out = len(specs)

    flat = []
    layout = []
    for sp in specs:
        layout.append((len(flat), len(sp) == 2))
        flat.extend(sp)

    def body(*refs):
        in_refs = refs[:len(flat)]
        out_refs = refs[len(flat):]
        i = pl.program_id(0)
        for (start, has_b), o_ref in zip(layout, out_refs):
            x = in_refs[start][...]
            if has_b:
                x = x - in_refs[start + 1][...]
            s1 = jnp.sum(x, axis=0, keepdims=True)
            s2 = jnp.sum(x * x, axis=0, keepdims=True)
            z = jnp.concatenate([s1, s2], axis=0)

            @pl.when(i == 0)
            def _(o_ref=o_ref, z=z):
                o_ref[...] = z

            @pl.when(i > 0)
            def _(o_ref=o_ref, z=z):
                o_ref[...] += z

    in_specs = [pl.BlockSpec((bm, a.shape[1]), lambda i: (i, 0)) for a in flat]
    out_shape = [jax.ShapeDtypeStruct((2, sp[0].shape[1]), jnp.float32)
                 for sp in specs]
    out_specs = [pl.BlockSpec((2, sp[0].shape[1]), lambda i: (0, 0))
                 for sp in specs]
    res = pl.pallas_call(body, grid=grid, in_specs=in_specs,
                         out_specs=out_specs, out_shape=out_shape)(*flat)
    return list(res)


def _wstats_call(x, weights, *, bm_target=1000):
    m, d = x.shape
    bm = _pick_bm(m, bm_target)
    grid = (m // bm,)
    nw = len(weights)

    def body(*refs):
        x_ref = refs[0]
        w_refs = refs[1:1 + nw]
        out_refs = refs[1 + nw:]
        i = pl.program_id(0)
        xv = x_ref[...]
        for w_ref, o_ref in zip(w_refs, out_refs):
            w = w_ref[...]
            s1 = jnp.sum(xv * w, axis=0, keepdims=True)
            s2 = jnp.sum(xv * xv * w, axis=0, keepdims=True)
            z = jnp.concatenate([s1, s2], axis=0)

            @pl.when(i == 0)
            def _(o_ref=o_ref, z=z):
                o_ref[...] = z

            @pl.when(i > 0)
            def _(o_ref=o_ref, z=z):
                o_ref[...] += z

    inputs = [x] + [w.reshape(m, 1) for w in weights]
    in_specs = ([pl.BlockSpec((bm, d), lambda i: (i, 0))] +
                [pl.BlockSpec((bm, 1), lambda i: (i, 0))] * nw)
    out_shape = [jax.ShapeDtypeStruct((2, d), jnp.float32)] * nw
    out_specs = [pl.BlockSpec((2, d), lambda i: (0, 0))] * nw
    return list(pl.pallas_call(body, grid=grid, in_specs=in_specs,
                               out_specs=out_specs,
                               out_shape=out_shape)(*inputs))


def _bn_affine(stats, m):
    mu = stats[0] / m
    var = stats[1] / m - mu * mu
    s = lax.rsqrt(var + _BN_EPS)
    return s, -mu * s


def _compose_affine(s_in, t_in, s_out, t_out):
    return s_in * s_out, t_in * s_out + t_out


def _affine_stats(stats, s, t, m):
    s1, s2 = stats[0], stats[1]
    return jnp.stack([s * s1 + m * t,
                      s * s * s2 + 2.0 * s * t * s1 + m * t * t])


def _sc_gather2(table0, idx0, idx1, table1=None):
    if table1 is None:
        table1 = table0
    e = idx0.shape[0]
    d = table0.shape[1]
    per_w = e // _NW
    r = 200 if d <= 256 else 40
    while per_w % r:
        r //= 5
    n_chunks = per_w // r
    mesh = plsc.VectorSubcoreMesh(core_axis_name="c", subcore_axis_name="s")

    @functools.partial(
        pl.kernel,
        out_type=[jax.ShapeDtypeStruct((e, d), jnp.float32)] * 2,
        mesh=mesh,
        scratch_types=[
            pltpu.VMEM((per_w,), jnp.int32),
            pltpu.VMEM((r, d), jnp.float32),
            pltpu.VMEM((r, d), jnp.float32),
            pltpu.SemaphoreType.DMA,
            pltpu.SemaphoreType.DMA,
        ],
    )
    def k(t0_hbm, t1_hbm, i0_hbm, i1_hbm, o0_hbm, o1_hbm, idx_v,
          buf0, buf1, sem0, sem1):
        wid = lax.axis_index("s") * 2 + lax.axis_index("c")
        base = wid * per_w
        for t_hbm, i_hbm, o_hbm in ((t0_hbm, i0_hbm, o0_hbm),
                                    (t1_hbm, i1_hbm, o1_hbm)):
            pltpu.sync_copy(i_hbm.at[pl.ds(base, per_w)], idx_v)

            def mk(c, buf, sem, t_hbm=t_hbm):
                return pltpu.make_async_copy(
                    t_hbm.at[idx_v.at[pl.ds(c * r, r)]], buf, sem)

            mk(0, buf0, sem0).start()

            def body(c, carry, o_hbm=o_hbm, mk=mk):
                @pl.when(c % 2 == 0)
                def _():
                    mk(c, buf0, sem0).wait()

                    @pl.when(c + 1 < n_chunks)
                    def _():
                        mk(c + 1, buf1, sem1).start()

                    pltpu.sync_copy(buf0, o_hbm.at[pl.ds(base + c * r, r)])

                @pl.when(c % 2 == 1)
                def _():
                    mk(c, buf1, sem1).wait()

                    @pl.when(c + 1 < n_chunks)
                    def _():
                        mk(c + 1, buf0, sem0).start()

                    pltpu.sync_copy(buf1, o_hbm.at[pl.ds(base + c * r, r)])

                return carry

            lax.fori_loop(0, n_chunks, body, 0)

    return k(table0, table1, idx0, idx1)


def _segment_max_affine(msg, dst, s, t, n_nodes):
    agg = jax.ops.segment_max(msg, dst, num_segments=n_nodes)
    return jnp.where(jnp.isfinite(agg), agg * s + t, 0.0)


def _mlp3_edge(arrs, terms_in, w1_list, p, *, e_rows, nsplit_out=1):
    b1 = p['lin1']['b']
    terms = [(ia, ib, s, t, w)
             for (ia, ib, s, t), w in zip(terms_in, w1_list)]
    y1, st1 = _linear_call(arrs, terms, b1, act='relu', want_stats=True)
    s1, t1 = _bn_affine(st1, e_rows)
    y2, st2 = _linear_call([y1], [(0, None, s1, t1, p['lin2']['W'])],
                           p['lin2']['b'], act='relu', want_stats=True)
    s2, t2 = _bn_affine(st2, e_rows)
    y3, st3 = _linear_call([y2], [(0, None, s2, t2, p['lin3']['W'])],
                           p['lin3']['b'], act='relu', want_stats=True,
                           nsplit=nsplit_out)
    s3, t3 = _bn_affine(st3, e_rows)
    return y3, (s3, t3), st3


def kernel(node_feats, edge_feats, params, edge_index):
    src = edge_index[0]
    dst = edge_index[1]
    n = node_feats.shape[0]
    e = src.shape[0]
    ef32 = jnp.float32(e)

    d0 = node_feats.shape[1]
    pad0 = (-d0) % 128
    nf = jnp.pad(node_feats, ((0, 0), (0, pad0)))
    w1n = params['nmm1']['lin1']['W']
    zpad = jnp.zeros((pad0, w1n.shape[1]), jnp.float32)
    w1n_parts = [jnp.concatenate([w1n[:d0], zpad]),
                 jnp.concatenate([w1n[d0:], zpad])]
    xd0, xs0 = _sc_gather2(nf, dst, src)
    st_a, st_b = _colstats_call([(xd0,), (xs0, xd0)])
    s0a, t0a = _bn_affine(st_a, ef32)
    s0b, t0b = _bn_affine(st_b, ef32)
    y3, (s3, t3), _ = _mlp3_edge(
        [xd0, xs0], [(0, None, s0a, t0a), (1, 0, s0b, t0b)], w1n_parts,
        params['nmm1'], e_rows=ef32)
    x1 = _segment_max_affine(y3, dst, s3, t3, n)

    xs1, xd1 = _sc_gather2(x1, src, dst)
    st_e0, st_s1, st_d1 = _colstats_call([(edge_feats,), (xs1,), (xd1,)])
    se0, te0 = _bn_affine(st_e0, ef32)
    ss1, ts1 = _bn_affine(st_s1, ef32)
    sd1, td1 = _bn_affine(st_d1, ef32)
    w1e = params['emm1']['lin1']['W']
    de0, d1 = edge_feats.shape[1], xs1.shape[1]
    e1, (es3, et3), est3 = _mlp3_edge(
        [edge_feats, xs1, xd1],
        [(0, None, se0, te0), (1, None, ss1, ts1), (2, None, sd1, td1)],
        [w1e[:de0], w1e[de0:de0 + d1], w1e[de0 + d1:]],
        params['emm1'], e_rows=ef32)

    w1n2 = params['nmm2']['lin1']['W']
    z3, (zs3, zt3), _ = _mlp3_edge(
        [xd1, xs1], [(0, None, None, None), (1, 0, None, None)],
        [w1n2[:d1], w1n2[d1:]], params['nmm2'], e_rows=ef32)
    x2 = _segment_max_affine(z3, dst, zs3, zt3, n)

    idx2 = jnp.concatenate([src, dst + n])
    cnt2 = jax.ops.segment_sum(jnp.ones((2 * e,), jnp.float32), idx2,
                               num_segments=2 * n)
    st_s2, st_d2 = _wstats_call(x2, [cnt2[:n], cnt2[n:]])
    st_e1p = _affine_stats(est3, es3, et3, ef32)
    se1, te1 = _bn_affine(st_e1p, ef32)
    se1c, te1c = _compose_affine(es3, et3, se1, te1)
    ss2, ts2 = _bn_affine(st_s2, ef32)
    sd2, td2 = _bn_affine(st_d2, ef32)
    w1e2 = params['emm2']['lin1']['W']
    de1, d2 = e1.shape[1], x2.shape[1]
    w_s, w_d = w1e2[de1:de1 + d2], w1e2[de1 + d2:]
    a_s, _ = _linear_call([x2], [(0, None, None, None, ss2[:, None] * w_s)],
                          ts2 @ w_s, act=None, want_stats=False,
                          bm_target=1000)
    a_d, _ = _linear_call([x2], [(0, None, None, None, sd2[:, None] * w_d)],
                          td2 @ w_d, act=None, want_stats=False,
                          bm_target=1000)
    as_g, ad_g = _sc_gather2(a_s, src, dst, table1=a_d)
    e2, (fs3, ft3), _ = _mlp3_edge(
        [e1, as_g, ad_g],
        [(0, None, se1c, te1c), (1, None, None, None),
         (2, None, None, None)],
        [w1e2[:de1], None, None],
        params['emm2'], e_rows=ef32)

    ph = params['nhead']
    h1, _ = _linear_call([x2], [(0, None, None, None, ph['l1']['W'])],
                         ph['l1']['b'], act='relu', want_stats=False,
                         bm_target=1000)
    h2, _ = _linear_call([h1], [(0, None, None, None, ph['l2']['W'])],
                         ph['l2']['b'], act='relu', want_stats=False,
                         bm_target=1000)
    w34 = ph['l3']['W'] @ ph['l4']['W']
    b34 = ph['l3']['b'] @ ph['l4']['W'] + ph['l4']['b']
    n_out, _ = _linear_call([h2], [(0, None, None, None, w34)], b34,
                            act='sigmoid', want_stats=False, bm_target=1000)

    pe = params['ehead']
    w12 = pe['l1']['W'] @ pe['l2']['W']
    b12 = pe['l1']['b'] @ pe['l2']['W'] + pe['l2']['b']
    w12f = fs3.reshape(-1, 1) * w12
    b12f = ft3 @ w12 + b12
    g1, _ = _linear_call([e2], [(0, None, None, None, w12f)], b12f,
                         act='relu', want_stats=False)
    g2, _ = _linear_call([g1], [(0, None, None, None, pe['l3']['W'])],
                         pe['l3']['b'], act='relu', want_stats=False)
    w45 = pe['l4']['W'] @ pe['l5']['W']
    b45 = pe['l4']['b'] @ pe['l5']['W'] + pe['l5']['b']
    e_out, _ = _linear_call([g2], [(0, None, None, None, w45)], b45,
                            act='sigmoid', want_stats=False)

    return (n_out, e_out)


# speedup vs baseline: 1.2121x; 1.1260x over previous
import functools
from typing import Sequence

import jax
import jax.numpy as jnp
from jax import lax
from jax.experimental import pallas as pl
from jax.experimental.pallas import tpu as pltpu
from jax.experimental.pallas import tpu_sc as plsc

_BN_EPS = 1e-5
_NW = 32


def _pick_bm(m, target):
    for bm in (target, 2048, 1600, 1280, 1024, 1000, 800, 640, 512, 400, 320,
               256, 200, 160, 128, 80, 64, 40, 32, 16, 8):
        if bm <= m and m % bm == 0 and bm % 8 == 0:
            return bm
    return m


def _linear_call(arrs, terms, b, *, act, want_stats, bm_target=1280,
                 nsplit=1, out_dtype=jnp.float32):
    m = arrs[0].shape[0]
    dout = terms[0][4].shape[1]
    dpart = dout // nsplit
    bm = _pick_bm(m, bm_target)
    grid = (m // bm,)

    n_arr = len(arrs)
    has_aff = [t[2] is not None for t in terms]

    def body(*refs):
        arr_refs = refs[:n_arr]
        k = n_arr
        term_data = []
        for (ia, ib, s, t, w), aff in zip(terms, has_aff):
            s_ref = t_ref = None
            if aff:
                s_ref, t_ref = refs[k], refs[k + 1]
                k += 2
            w_ref = None
            if w is not None:
                w_ref = refs[k]
                k += 1
            term_data.append((ia, ib, s_ref, t_ref, w_ref))
        b_ref = refs[k]
        k += 1
        out_refs = refs[k:k + nsplit]
        st_ref = refs[k + nsplit] if want_stats else None

        acc = jnp.zeros((bm, dout), jnp.float32) + b_ref[...]
        for (ia, ib, s_ref, t_ref, w_ref) in term_data:
            x = arr_refs[ia][...]
            if ib is not None:
                x = x - arr_refs[ib][...]
            if s_ref is not None:
                x = x * s_ref[...] + t_ref[...]
            if w_ref is None:
                acc = acc + x
            else:
                acc = acc + jnp.dot(x, w_ref[...],
                                    preferred_element_type=jnp.float32)
        if act == 'relu':
            acc = jnp.maximum(acc, 0.0)
        elif act == 'sigmoid':
            acc = jax.nn.sigmoid(acc)
        for p_i, o_ref in enumerate(out_refs):
            o_ref[...] = acc[:, p_i * dpart:(p_i + 1) * dpart].astype(
                out_dtype)
        if want_stats:
            s1 = jnp.sum(acc, axis=0, keepdims=True)
            s2 = jnp.sum(acc * acc, axis=0, keepdims=True)
            z = jnp.concatenate([s1, s2], axis=0)
            i = pl.program_id(0)

            @pl.when(i == 0)
            def _():
                st_ref[...] = z

            @pl.when(i > 0)
            def _():
                st_ref[...] += z

    in_specs = []
    inputs = []
    for a in arrs:
        inputs.append(a)
        in_specs.append(pl.BlockSpec((bm, a.shape[1]), lambda i: (i, 0)))
    for (ia, ib, s, t, w), aff in zip(terms, has_aff):
        din = w.shape[0] if w is not None else arrs[ia].shape[1]
        if aff:
            inputs += [s.reshape(1, din), t.reshape(1, din)]
            in_specs += [pl.BlockSpec((1, din), lambda i: (0, 0))] * 2
        if w is not None:
            inputs.append(w)
            in_specs.append(pl.BlockSpec((din, dout), lambda i: (0, 0)))
    inputs.append(b.reshape(1, dout))
    in_specs.append(pl.BlockSpec((1, dout), lambda i: (0, 0)))

    out_shape = [jax.ShapeDtypeStruct((m, dpart), out_dtype)] * nsplit
    out_specs = [pl.BlockSpec((bm, dpart), lambda i: (i, 0))] * nsplit
    if want_stats:
        out_shape.append(jax.ShapeDtypeStruct((2, dout), jnp.float32))
        out_specs.append(pl.BlockSpec((2, dout), lambda i: (0, 0)))

    res = pl.pallas_call(
        body, grid=grid, in_specs=in_specs, out_specs=out_specs,
        out_shape=out_shape)(*inputs)
    outs = res[0] if nsplit == 1 else list(res[:nsplit])
    return (outs, res[nsplit]) if want_stats else (outs, None)


def _colstats_call(specs, *, bm_target=1280):
    m = specs[0][0].shape[0]
    bm = _pick_bm(m, bm_target)
    grid = (m // bm,)
    n_out = len(specs)

    flat = []
    layout = []
    for sp in specs:
        layout.append((len(flat), len(sp) == 2))
        flat.extend(sp)

    def body(*refs):
        in_refs = refs[:len(flat)]
        out_refs = refs[len(flat):]
        i = pl.program_id(0)
        for (start, has_b), o_ref in zip(layout, out_refs):
            x = in_refs[start][...]
            if has_b:
                x = x - in_refs[start + 1][...]
            s1 = jnp.sum(x, axis=0, keepdims=True)
            s2 = jnp.sum(x * x, axis=0, keepdims=True)
            z = jnp.concatenate([s1, s2], axis=0)

            @pl.when(i == 0)
            def _(o_ref=o_ref, z=z):
                o_ref[...] = z

            @pl.when(i > 0)
            def _(o_ref=o_ref, z=z):
                o_ref[...] += z

    in_specs = [pl.BlockSpec((bm, a.shape[1]), lambda i: (i, 0)) for a in flat]
    out_shape = [jax.ShapeDtypeStruct((2, sp[0].shape[1]), jnp.float32)
                 for sp in specs]
    out_specs = [pl.BlockSpec((2, sp[0].shape[1]), lambda i: (0, 0))
                 for sp in specs]
    res = pl.pallas_call(body, grid=grid, in_specs=in_specs,
                         out_specs=out_specs, out_shape=out_shape)(*flat)
    return list(res)


def _wstats_call(x, weights, *, bm_target=1000):
    m, d = x.shape
    bm = _pick_bm(m, bm_target)
    grid = (m // bm,)
    nw = len(weights)

    def body(*refs):
        x_ref = refs[0]
        w_refs = refs[1:1 + nw]
        out_refs = refs[1 + nw:]
        i = pl.program_id(0)
        xv = x_ref[...]
        for w_ref, o_ref in zip(w_refs, out_refs):
            w = w_ref[...]
            s1 = jnp.sum(xv * w, axis=0, keepdims=True)
            s2 = jnp.sum(xv * xv * w, axis=0, keepdims=True)
            z = jnp.concatenate([s1, s2], axis=0)

            @pl.when(i == 0)
            def _(o_ref=o_ref, z=z):
                o_ref[...] = z

            @pl.when(i > 0)
            def _(o_ref=o_ref, z=z):
                o_ref[...] += z

    inputs = [x] + [w.reshape(m, 1) for w in weights]
    in_specs = ([pl.BlockSpec((bm, d), lambda i: (i, 0))] +
                [pl.BlockSpec((bm, 1), lambda i: (i, 0))] * nw)
    out_shape = [jax.ShapeDtypeStruct((2, d), jnp.float32)] * nw
    out_specs = [pl.BlockSpec((2, d), lambda i: (0, 0))] * nw
    return list(pl.pallas_call(body, grid=grid, in_specs=in_specs,
                               out_specs=out_specs,
                               out_shape=out_shape)(*inputs))


def _bn_affine(stats, m):
    mu = stats[0] / m
    var = stats[1] / m - mu * mu
    s = lax.rsqrt(var + _BN_EPS)
    return s, -mu * s


def _compose_affine(s_in, t_in, s_out, t_out):
    return s_in * s_out, t_in * s_out + t_out


def _affine_stats(stats, s, t, m):
    s1, s2 = stats[0], stats[1]
    return jnp.stack([s * s1 + m * t,
                      s * s * s2 + 2.0 * s * t * s1 + m * t * t])


def _sc_gather2(table0, idx0, idx1, table1=None):
    if table1 is None:
        table1 = table0
    e = idx0.shape[0]
    d = table0.shape[1]
    per_w = e // _NW
    r = 200 if d <= 256 else 40
    while per_w % r:
        r //= 5
    n_chunks = per_w // r
    mesh = plsc.VectorSubcoreMesh(core_axis_name="c", subcore_axis_name="s")

    @functools.partial(
        pl.kernel,
        out_type=[jax.ShapeDtypeStruct((e, d), jnp.float32)] * 2,
        mesh=mesh,
        scratch_types=[
            pltpu.VMEM((per_w,), jnp.int32),
            pltpu.VMEM((r, d), jnp.float32),
            pltpu.VMEM((r, d), jnp.float32),
            pltpu.SemaphoreType.DMA,
            pltpu.SemaphoreType.DMA,
        ],
    )
    def k(t0_hbm, t1_hbm, i0_hbm, i1_hbm, o0_hbm, o1_hbm, idx_v,
          buf0, buf1, sem0, sem1):
        wid = lax.axis_index("s") * 2 + lax.axis_index("c")
        base = wid * per_w
        for t_hbm, i_hbm, o_hbm in ((t0_hbm, i0_hbm, o0_hbm),
                                    (t1_hbm, i1_hbm, o1_hbm)):
            pltpu.sync_copy(i_hbm.at[pl.ds(base, per_w)], idx_v)

            def mk(c, buf, sem, t_hbm=t_hbm):
                return pltpu.make_async_copy(
                    t_hbm.at[idx_v.at[pl.ds(c * r, r)]], buf, sem)

            mk(0, buf0, sem0).start()

            def body(c, carry, o_hbm=o_hbm, mk=mk):
                @pl.when(c % 2 == 0)
                def _():
                    mk(c, buf0, sem0).wait()

                    @pl.when(c + 1 < n_chunks)
                    def _():
                        mk(c + 1, buf1, sem1).start()

                    pltpu.sync_copy(buf0, o_hbm.at[pl.ds(base + c * r, r)])

                @pl.when(c % 2 == 1)
                def _():
                    mk(c, buf1, sem1).wait()

                    @pl.when(c + 1 < n_chunks)
                    def _():
                        mk(c + 1, buf0, sem0).start()

                    pltpu.sync_copy(buf1, o_hbm.at[pl.ds(base + c * r, r)])

                return carry

            lax.fori_loop(0, n_chunks, body, 0)

    return k(table0, table1, idx0, idx1)


def _segment_max_affine(msg, dst, s, t, n_nodes):
    agg = jax.ops.segment_max(msg, dst, num_segments=n_nodes)
    agg32 = agg.astype(jnp.float32)
    return jnp.where(jnp.isfinite(agg), agg32 * s + t, 0.0)


def _mlp3_edge(arrs, terms_in, w1_list, p, *, e_rows, nsplit_out=1,
               out_dtype=jnp.float32):
    b1 = p['lin1']['b']
    terms = [(ia, ib, s, t, w)
             for (ia, ib, s, t), w in zip(terms_in, w1_list)]
    y1, st1 = _linear_call(arrs, terms, b1, act='relu', want_stats=True)
    s1, t1 = _bn_affine(st1, e_rows)
    y2, st2 = _linear_call([y1], [(0, None, s1, t1, p['lin2']['W'])],
                           p['lin2']['b'], act='relu', want_stats=True)
    s2, t2 = _bn_affine(st2, e_rows)
    y3, st3 = _linear_call([y2], [(0, None, s2, t2, p['lin3']['W'])],
                           p['lin3']['b'], act='relu', want_stats=True,
                           nsplit=nsplit_out, out_dtype=out_dtype)
    s3, t3 = _bn_affine(st3, e_rows)
    return y3, (s3, t3), st3


def kernel(node_feats, edge_feats, params, edge_index):
    src = edge_index[0]
    dst = edge_index[1]
    n = node_feats.shape[0]
    e = src.shape[0]
    ef32 = jnp.float32(e)

    d0 = node_feats.shape[1]
    pad0 = (-d0) % 128
    nf = jnp.pad(node_feats, ((0, 0), (0, pad0)))
    w1n = params['nmm1']['lin1']['W']
    zpad = jnp.zeros((pad0, w1n.shape[1]), jnp.float32)
    w1n_parts = [jnp.concatenate([w1n[:d0], zpad]),
                 jnp.concatenate([w1n[d0:], zpad])]
    xd0, xs0 = _sc_gather2(nf, dst, src)
    st_a, st_b = _colstats_call([(xd0,), (xs0, xd0)])
    s0a, t0a = _bn_affine(st_a, ef32)
    s0b, t0b = _bn_affine(st_b, ef32)
    y3, (s3, t3), _ = _mlp3_edge(
        [xd0, xs0], [(0, None, s0a, t0a), (1, 0, s0b, t0b)], w1n_parts,
        params['nmm1'], e_rows=ef32, out_dtype=jnp.bfloat16)
    x1 = _segment_max_affine(y3, dst, s3, t3, n)

    xs1, xd1 = _sc_gather2(x1, src, dst)
    st_e0, st_s1, st_d1 = _colstats_call([(edge_feats,), (xs1,), (xd1,)])
    se0, te0 = _bn_affine(st_e0, ef32)
    ss1, ts1 = _bn_affine(st_s1, ef32)
    sd1, td1 = _bn_affine(st_d1, ef32)
    w1e = params['emm1']['lin1']['W']
    de0, d1 = edge_feats.shape[1], xs1.shape[1]
    e1, (es3, et3), est3 = _mlp3_edge(
        [edge_feats, xs1, xd1],
        [(0, None, se0, te0), (1, None, ss1, ts1), (2, None, sd1, td1)],
        [w1e[:de0], w1e[de0:de0 + d1], w1e[de0 + d1:]],
        params['emm1'], e_rows=ef32)

    w1n2 = params['nmm2']['lin1']['W']
    z3, (zs3, zt3), _ = _mlp3_edge(
        [xd1, xs1], [(0, None, None, None), (1, 0, None, None)],
        [w1n2[:d1], w1n2[d1:]], params['nmm2'], e_rows=ef32,
        out_dtype=jnp.bfloat16)
    x2 = _segment_max_affine(z3, dst, zs3, zt3, n)

    idx2 = jnp.concatenate([src, dst + n])
    cnt2 = jax.ops.segment_sum(jnp.ones((2 * e,), jnp.float32), idx2,
                               num_segments=2 * n)
    st_s2, st_d2 = _wstats_call(x2, [cnt2[:n], cnt2[n:]])
    st_e1p = _affine_stats(est3, es3, et3, ef32)
    se1, te1 = _bn_affine(st_e1p, ef32)
    se1c, te1c = _compose_affine(es3, et3, se1, te1)
    ss2, ts2 = _bn_affine(st_s2, ef32)
    sd2, td2 = _bn_affine(st_d2, ef32)
    w1e2 = params['emm2']['lin1']['W']
    de1, d2 = e1.shape[1], x2.shape[1]
    w_s, w_d = w1e2[de1:de1 + d2], w1e2[de1 + d2:]
    a_s, _ = _linear_call([x2], [(0, None, None, None, ss2[:, None] * w_s)],
                          ts2 @ w_s, act=None, want_stats=False,
                          bm_target=1000)
    a_d, _ = _linear_call([x2], [(0, None, None, None, sd2[:, None] * w_d)],
                          td2 @ w_d, act=None, want_stats=False,
                          bm_target=1000)
    as_g, ad_g = _sc_gather2(a_s, src, dst, table1=a_d)
    e2, (fs3, ft3), _ = _mlp3_edge(
        [e1, as_g, ad_g],
        [(0, None, se1c, te1c), (1, None, None, None),
         (2, None, None, None)],
        [w1e2[:de1], None, None],
        params['emm2'], e_rows=ef32)

    ph = params['nhead']
    h1, _ = _linear_call([x2], [(0, None, None, None, ph['l1']['W'])],
                         ph['l1']['b'], act='relu', want_stats=False,
                         bm_target=1000)
    h2, _ = _linear_call([h1], [(0, None, None, None, ph['l2']['W'])],
                         ph['l2']['b'], act='relu', want_stats=False,
                         bm_target=1000)
    w34 = ph['l3']['W'] @ ph['l4']['W']
    b34 = ph['l3']['b'] @ ph['l4']['W'] + ph['l4']['b']
    n_out, _ = _linear_call([h2], [(0, None, None, None, w34)], b34,
                            act='sigmoid', want_stats=False, bm_target=1000)

    pe = params['ehead']
    w12 = pe['l1']['W'] @ pe['l2']['W']
    b12 = pe['l1']['b'] @ pe['l2']['W'] + pe['l2']['b']
    w12f = fs3.reshape(-1, 1) * w12
    b12f = ft3 @ w12 + b12
    g1, _ = _linear_call([e2], [(0, None, None, None, w12f)], b12f,
                         act='relu', want_stats=False)
    g2, _ = _linear_call([g1], [(0, None, None, None, pe['l3']['W'])],
                         pe['l3']['b'], act='relu', want_stats=False)
    w45 = pe['l4']['W'] @ pe['l5']['W']
    b45 = pe['l4']['b'] @ pe['l5']['W'] + pe['l5']['b']
    e_out, _ = _linear_call([g2], [(0, None, None, None, w45)], b45,
                            act='sigmoid', want_stats=False)

    return (n_out, e_out)


# speedup vs baseline: 1.2154x; 1.0027x over previous
import functools
from typing import Sequence

import jax
import jax.numpy as jnp
from jax import lax
from jax.experimental import pallas as pl
from jax.experimental.pallas import tpu as pltpu
from jax.experimental.pallas import tpu_sc as plsc

_BN_EPS = 1e-5
_NW = 32


def _pick_bm(m, target):
    for bm in (target, 2048, 1600, 1280, 1024, 1000, 800, 640, 512, 400, 320,
               256, 200, 160, 128, 80, 64, 40, 32, 16, 8):
        if bm <= m and m % bm == 0 and bm % 8 == 0:
            return bm
    return m


def _linear_call(arrs, terms, b, *, act, want_stats, bm_target=1280,
                 nsplit=1, out_dtype=jnp.float32, mxu_bf16=False):
    m = arrs[0].shape[0]
    dout = terms[0][4].shape[1]
    dpart = dout // nsplit
    bm = _pick_bm(m, bm_target)
    grid = (m // bm,)

    n_arr = len(arrs)
    has_aff = [t[2] is not None for t in terms]

    def body(*refs):
        arr_refs = refs[:n_arr]
        k = n_arr
        term_data = []
        for (ia, ib, s, t, w), aff in zip(terms, has_aff):
            s_ref = t_ref = None
            if aff:
                s_ref, t_ref = refs[k], refs[k + 1]
                k += 2
            w_ref = None
            if w is not None:
                w_ref = refs[k]
                k += 1
            term_data.append((ia, ib, s_ref, t_ref, w_ref))
        b_ref = refs[k]
        k += 1
        out_refs = refs[k:k + nsplit]
        st_ref = refs[k + nsplit] if want_stats else None

        acc = jnp.zeros((bm, dout), jnp.float32) + b_ref[...]
        for (ia, ib, s_ref, t_ref, w_ref) in term_data:
            x = arr_refs[ia][...]
            if ib is not None:
                x = x - arr_refs[ib][...]
            if s_ref is not None:
                x = x * s_ref[...] + t_ref[...]
            if w_ref is None:
                acc = acc + x
            elif mxu_bf16:
                acc = acc + jnp.dot(x.astype(jnp.bfloat16),
                                    w_ref[...].astype(jnp.bfloat16),
                                    preferred_element_type=jnp.float32)
            else:
                acc = acc + jnp.dot(x, w_ref[...],
                                    preferred_element_type=jnp.float32)
        if act == 'relu':
            acc = jnp.maximum(acc, 0.0)
        elif act == 'sigmoid':
            acc = jax.nn.sigmoid(acc)
        for p_i, o_ref in enumerate(out_refs):
            o_ref[...] = acc[:, p_i * dpart:(p_i + 1) * dpart].astype(
                out_dtype)
        if want_stats:
            s1 = jnp.sum(acc, axis=0, keepdims=True)
            s2 = jnp.sum(acc * acc, axis=0, keepdims=True)
            z = jnp.concatenate([s1, s2], axis=0)
            i = pl.program_id(0)

            @pl.when(i == 0)
            def _():
                st_ref[...] = z

            @pl.when(i > 0)
            def _():
                st_ref[...] += z

    in_specs = []
    inputs = []
    for a in arrs:
        inputs.append(a)
        in_specs.append(pl.BlockSpec((bm, a.shape[1]), lambda i: (i, 0)))
    for (ia, ib, s, t, w), aff in zip(terms, has_aff):
        din = w.shape[0] if w is not None else arrs[ia].shape[1]
        if aff:
            inputs += [s.reshape(1, din), t.reshape(1, din)]
            in_specs += [pl.BlockSpec((1, din), lambda i: (0, 0))] * 2
        if w is not None:
            inputs.append(w)
            in_specs.append(pl.BlockSpec((din, dout), lambda i: (0, 0)))
    inputs.append(b.reshape(1, dout))
    in_specs.append(pl.BlockSpec((1, dout), lambda i: (0, 0)))

    out_shape = [jax.ShapeDtypeStruct((m, dpart), out_dtype)] * nsplit
    out_specs = [pl.BlockSpec((bm, dpart), lambda i: (i, 0))] * nsplit
    if want_stats:
        out_shape.append(jax.ShapeDtypeStruct((2, dout), jnp.float32))
        out_specs.append(pl.BlockSpec((2, dout), lambda i: (0, 0)))

    res = pl.pallas_call(
        body, grid=grid, in_specs=in_specs, out_specs=out_specs,
        out_shape=out_shape)(*inputs)
    outs = res[0] if nsplit == 1 else list(res[:nsplit])
    return (outs, res[nsplit]) if want_stats else (outs, None)


def _colstats_call(specs, *, bm_target=1280):
    m = specs[0][0].shape[0]
    bm = _pick_bm(m, bm_target)
    grid = (m // bm,)
    n_out = len(specs)

    flat = []
    layout = []
    for sp in specs:
        layout.append((len(flat), len(sp) == 2))
        flat.extend(sp)

    def body(*refs):
        in_refs = refs[:len(flat)]
        out_refs = refs[len(flat):]
        i = pl.program_id(0)
        for (start, has_b), o_ref in zip(layout, out_refs):
            x = in_refs[start][...]
            if has_b:
                x = x - in_refs[start + 1][...]
            s1 = jnp.sum(x, axis=0, keepdims=True)
            s2 = jnp.sum(x * x, axis=0, keepdims=True)
            z = jnp.concatenate([s1, s2], axis=0)

            @pl.when(i == 0)
            def _(o_ref=o_ref, z=z):
                o_ref[...] = z

            @pl.when(i > 0)
            def _(o_ref=o_ref, z=z):
                o_ref[...] += z

    in_specs = [pl.BlockSpec((bm, a.shape[1]), lambda i: (i, 0)) for a in flat]
    out_shape = [jax.ShapeDtypeStruct((2, sp[0].shape[1]), jnp.float32)
                 for sp in specs]
    out_specs = [pl.BlockSpec((2, sp[0].shape[1]), lambda i: (0, 0))
                 for sp in specs]
    res = pl.pallas_call(body, grid=grid, in_specs=in_specs,
                         out_specs=out_specs, out_shape=out_shape)(*flat)
    return list(res)


def _wstats_call(x, weights, *, bm_target=1000):
    m, d = x.shape
    bm = _pick_bm(m, bm_target)
    grid = (m // bm,)
    nw = len(weights)

    def body(*refs):
        x_ref = refs[0]
        w_refs = refs[1:1 + nw]
        out_refs = refs[1 + nw:]
        i = pl.program_id(0)
        xv = x_ref[...]
        for w_ref, o_ref in zip(w_refs, out_refs):
            w = w_ref[...]
            s1 = jnp.sum(xv * w, axis=0, keepdims=True)
            s2 = jnp.sum(xv * xv * w, axis=0, keepdims=True)
            z = jnp.concatenate([s1, s2], axis=0)

            @pl.when(i == 0)
            def _(o_ref=o_ref, z=z):
                o_ref[...] = z

            @pl.when(i > 0)
            def _(o_ref=o_ref, z=z):
                o_ref[...] += z

    inputs = [x] + [w.reshape(m, 1) for w in weights]
    in_specs = ([pl.BlockSpec((bm, d), lambda i: (i, 0))] +
                [pl.BlockSpec((bm, 1), lambda i: (i, 0))] * nw)
    out_shape = [jax.ShapeDtypeStruct((2, d), jnp.float32)] * nw
    out_specs = [pl.BlockSpec((2, d), lambda i: (0, 0))] * nw
    return list(pl.pallas_call(body, grid=grid, in_specs=in_specs,
                               out_specs=out_specs,
                               out_shape=out_shape)(*inputs))


def _bn_affine(stats, m):
    mu = stats[0] / m
    var = stats[1] / m - mu * mu
    s = lax.rsqrt(var + _BN_EPS)
    return s, -mu * s


def _compose_affine(s_in, t_in, s_out, t_out):
    return s_in * s_out, t_in * s_out + t_out


def _affine_stats(stats, s, t, m):
    s1, s2 = stats[0], stats[1]
    return jnp.stack([s * s1 + m * t,
                      s * s * s2 + 2.0 * s * t * s1 + m * t * t])


def _sc_gather2(table0, idx0, idx1, table1=None):
    if table1 is None:
        table1 = table0
    e = idx0.shape[0]
    d = table0.shape[1]
    per_w = e // _NW
    r = 200 if d <= 256 else 40
    while per_w % r:
        r //= 5
    n_chunks = per_w // r
    mesh = plsc.VectorSubcoreMesh(core_axis_name="c", subcore_axis_name="s")

    @functools.partial(
        pl.kernel,
        out_type=[jax.ShapeDtypeStruct((e, d), jnp.float32)] * 2,
        mesh=mesh,
        scratch_types=[
            pltpu.VMEM((per_w,), jnp.int32),
            pltpu.VMEM((r, d), jnp.float32),
            pltpu.VMEM((r, d), jnp.float32),
            pltpu.SemaphoreType.DMA,
            pltpu.SemaphoreType.DMA,
        ],
    )
    def k(t0_hbm, t1_hbm, i0_hbm, i1_hbm, o0_hbm, o1_hbm, idx_v,
          buf0, buf1, sem0, sem1):
        wid = lax.axis_index("s") * 2 + lax.axis_index("c")
        base = wid * per_w
        for t_hbm, i_hbm, o_hbm in ((t0_hbm, i0_hbm, o0_hbm),
                                    (t1_hbm, i1_hbm, o1_hbm)):
            pltpu.sync_copy(i_hbm.at[pl.ds(base, per_w)], idx_v)

            def mk(c, buf, sem, t_hbm=t_hbm):
                return pltpu.make_async_copy(
                    t_hbm.at[idx_v.at[pl.ds(c * r, r)]], buf, sem)

            mk(0, buf0, sem0).start()

            def body(c, carry, o_hbm=o_hbm, mk=mk):
                @pl.when(c % 2 == 0)
                def _():
                    mk(c, buf0, sem0).wait()

                    @pl.when(c + 1 < n_chunks)
                    def _():
                        mk(c + 1, buf1, sem1).start()

                    pltpu.sync_copy(buf0, o_hbm.at[pl.ds(base + c * r, r)])

                @pl.when(c % 2 == 1)
                def _():
                    mk(c, buf1, sem1).wait()

                    @pl.when(c + 1 < n_chunks)
                    def _():
                        mk(c + 1, buf0, sem0).start()

                    pltpu.sync_copy(buf1, o_hbm.at[pl.ds(base + c * r, r)])

                return carry

            lax.fori_loop(0, n_chunks, body, 0)

    return k(table0, table1, idx0, idx1)


def _segment_max_affine(msg, dst, s, t, n_nodes):
    agg = jax.ops.segment_max(msg, dst, num_segments=n_nodes)
    agg32 = agg.astype(jnp.float32)
    return jnp.where(jnp.isfinite(agg), agg32 * s + t, 0.0)


def _mlp3_edge(arrs, terms_in, w1_list, p, *, e_rows, nsplit_out=1,
               out_dtype=jnp.float32, mxu_bf16=False):
    mid_dtype = jnp.bfloat16 if mxu_bf16 else jnp.float32
    b1 = p['lin1']['b']
    terms = [(ia, ib, s, t, w)
             for (ia, ib, s, t), w in zip(terms_in, w1_list)]
    y1, st1 = _linear_call(arrs, terms, b1, act='relu', want_stats=True,
                           out_dtype=mid_dtype, mxu_bf16=mxu_bf16)
    s1, t1 = _bn_affine(st1, e_rows)
    y2, st2 = _linear_call([y1], [(0, None, s1, t1, p['lin2']['W'])],
                           p['lin2']['b'], act='relu', want_stats=True,
                           out_dtype=mid_dtype, mxu_bf16=mxu_bf16)
    s2, t2 = _bn_affine(st2, e_rows)
    y3, st3 = _linear_call([y2], [(0, None, s2, t2, p['lin3']['W'])],
                           p['lin3']['b'], act='relu', want_stats=True,
                           nsplit=nsplit_out, out_dtype=out_dtype,
                           mxu_bf16=mxu_bf16)
    s3, t3 = _bn_affine(st3, e_rows)
    return y3, (s3, t3), st3


def kernel(node_feats, edge_feats, params, edge_index):
    src = edge_index[0]
    dst = edge_index[1]
    n = node_feats.shape[0]
    e = src.shape[0]
    ef32 = jnp.float32(e)

    d0 = node_feats.shape[1]
    pad0 = (-d0) % 128
    nf = jnp.pad(node_feats, ((0, 0), (0, pad0)))
    w1n = params['nmm1']['lin1']['W']
    zpad = jnp.zeros((pad0, w1n.shape[1]), jnp.float32)
    w1n_parts = [jnp.concatenate([w1n[:d0], zpad]),
                 jnp.concatenate([w1n[d0:], zpad])]
    xd0, xs0 = _sc_gather2(nf, dst, src)
    st_a, st_b = _colstats_call([(xd0,), (xs0, xd0)])
    s0a, t0a = _bn_affine(st_a, ef32)
    s0b, t0b = _bn_affine(st_b, ef32)
    y3, (s3, t3), _ = _mlp3_edge(
        [xd0, xs0], [(0, None, s0a, t0a), (1, 0, s0b, t0b)], w1n_parts,
        params['nmm1'], e_rows=ef32, out_dtype=jnp.bfloat16)
    x1 = _segment_max_affine(y3, dst, s3, t3, n)

    xs1, xd1 = _sc_gather2(x1, src, dst)
    st_e0, st_s1, st_d1 = _colstats_call([(edge_feats,), (xs1,), (xd1,)])
    se0, te0 = _bn_affine(st_e0, ef32)
    ss1, ts1 = _bn_affine(st_s1, ef32)
    sd1, td1 = _bn_affine(st_d1, ef32)
    w1e = params['emm1']['lin1']['W']
    de0, d1 = edge_feats.shape[1], xs1.shape[1]
    e1, (es3, et3), est3 = _mlp3_edge(
        [edge_feats, xs1, xd1],
        [(0, None, se0, te0), (1, None, ss1, ts1), (2, None, sd1, td1)],
        [w1e[:de0], w1e[de0:de0 + d1], w1e[de0 + d1:]],
        params['emm1'], e_rows=ef32)

    w1n2 = params['nmm2']['lin1']['W']
    z3, (zs3, zt3), _ = _mlp3_edge(
        [xd1, xs1], [(0, None, None, None), (1, 0, None, None)],
        [w1n2[:d1], w1n2[d1:]], params['nmm2'], e_rows=ef32,
        out_dtype=jnp.bfloat16)
    x2 = _segment_max_affine(z3, dst, zs3, zt3, n)

    idx2 = jnp.concatenate([src, dst + n])
    cnt2 = jax.ops.segment_sum(jnp.ones((2 * e,), jnp.float32), idx2,
                               num_segments=2 * n)
    st_s2, st_d2 = _wstats_call(x2, [cnt2[:n], cnt2[n:]])
    st_e1p = _affine_stats(est3, es3, et3, ef32)
    se1, te1 = _bn_affine(st_e1p, ef32)
    se1c, te1c = _compose_affine(es3, et3, se1, te1)
    ss2, ts2 = _bn_affine(st_s2, ef32)
    sd2, td2 = _bn_affine(st_d2, ef32)
    w1e2 = params['emm2']['lin1']['W']
    de1, d2 = e1.shape[1], x2.shape[1]
    w_s, w_d = w1e2[de1:de1 + d2], w1e2[de1 + d2:]
    a_s, _ = _linear_call([x2], [(0, None, None, None, ss2[:, None] * w_s)],
                          ts2 @ w_s, act=None, want_stats=False,
                          bm_target=1000)
    a_d, _ = _linear_call([x2], [(0, None, None, None, sd2[:, None] * w_d)],
                          td2 @ w_d, act=None, want_stats=False,
                          bm_target=1000)
    as_g, ad_g = _sc_gather2(a_s, src, dst, table1=a_d)
    e2, (fs3, ft3), _ = _mlp3_edge(
        [e1, as_g, ad_g],
        [(0, None, se1c, te1c), (1, None, None, None),
         (2, None, None, None)],
        [w1e2[:de1], None, None],
        params['emm2'], e_rows=ef32)

    ph = params['nhead']
    h1, _ = _linear_call([x2], [(0, None, None, None, ph['l1']['W'])],
                         ph['l1']['b'], act='relu', want_stats=False,
                         bm_target=1000)
    h2, _ = _linear_call([h1], [(0, None, None, None, ph['l2']['W'])],
                         ph['l2']['b'], act='relu', want_stats=False,
                         bm_target=1000)
    w34 = ph['l3']['W'] @ ph['l4']['W']
    b34 = ph['l3']['b'] @ ph['l4']['W'] + ph['l4']['b']
    n_out, _ = _linear_call([h2], [(0, None, None, None, w34)], b34,
                            act='sigmoid', want_stats=False, bm_target=1000)

    pe = params['ehead']
    w12 = pe['l1']['W'] @ pe['l2']['W']
    b12 = pe['l1']['b'] @ pe['l2']['W'] + pe['l2']['b']
    w12f = fs3.reshape(-1, 1) * w12
    b12f = ft3 @ w12 + b12
    g1, _ = _linear_call([e2], [(0, None, None, None, w12f)], b12f,
                         act='relu', want_stats=False)
    g2, _ = _linear_call([g1], [(0, None, None, None, pe['l3']['W'])],
                         pe['l3']['b'], act='relu', want_stats=False)
    w45 = pe['l4']['W'] @ pe['l5']['W']
    b45 = pe['l4']['b'] @ pe['l5']['W'] + pe['l5']['b']
    e_out, _ = _linear_call([g2], [(0, None, None, None, w45)], b45,
                            act='sigmoid', want_stats=False)

    return (n_out, e_out)
